# Initial kernel scaffold; baseline (speedup 1.0000x reference)
import jax
import jax.numpy as jnp
from jax.experimental import pallas as pl


def kernel(x, expert_w, expert_b, router_w, router_b):
    raise NotImplementedError("write your pallas kernel here")


# speedup vs baseline: 1.0932x; 1.0932x over previous
import functools

import jax
import jax.numpy as jnp
from jax.experimental import pallas as pl
from jax.experimental.pallas import tpu as pltpu

N_TOKENS = 4096
D_MODEL = 1024
N_EXPERT = 8
N_SPARSE = N_EXPERT - 1
TOP_K = 2
BT = 256
NTB = N_TOKENS // BT
LANE = 128


def _moe_block_kernel(x_ref, w_ref, b_ref, rw_ref, rb_ref, out_ref, acc_ref, cmb_ref):
    e = pl.program_id(1)
    x = x_ref[...]

    @pl.when(e == 0)
    def _shared_and_route():
        acc_ref[...] = jnp.dot(x, w_ref[0], preferred_element_type=jnp.float32) + b_ref[0]
        logits = jnp.dot(x, rw_ref[...], preferred_element_type=jnp.float32)
        lane = jax.lax.broadcasted_iota(jnp.int32, (BT, LANE), 1)
        valid = lane < N_SPARSE
        logits = jnp.where(valid, logits + rb_ref[0][None, :], -1e30)
        m = jnp.max(logits, axis=1, keepdims=True)
        ex = jnp.where(valid, jnp.exp(logits - m), 0.0)
        p = ex / jnp.sum(ex, axis=1, keepdims=True)
        m1 = jnp.max(p, axis=1, keepdims=True)
        a1 = jnp.min(jnp.where(p == m1, lane, LANE), axis=1, keepdims=True)
        p2 = jnp.where(lane == a1, -1.0, p)
        m2 = jnp.max(p2, axis=1, keepdims=True)
        a2 = jnp.min(jnp.where(p2 == m2, lane, LANE), axis=1, keepdims=True)
        den = m1 + m2 + 1e-6
        w1 = m1 / den
        w2 = m2 / den
        cmb_ref[...] = jnp.where(lane == a1, w1, 0.0) + jnp.where(lane == a2, w2, 0.0)

    @pl.when(e > 0)
    def _sparse():
        lane = jax.lax.broadcasted_iota(jnp.int32, (BT, LANE), 1)
        wcol = jnp.sum(jnp.where(lane == e - 1, cmb_ref[...], 0.0), axis=1, keepdims=True)
        mm = jnp.dot(x, w_ref[0], preferred_element_type=jnp.float32) + b_ref[0]
        acc_ref[...] += wcol * mm

    @pl.when(e == N_EXPERT - 1)
    def _write():
        out_ref[...] = acc_ref[...]


def kernel(x, expert_w, expert_b, router_w, router_b):
    rw = jnp.zeros((D_MODEL, LANE), jnp.float32).at[:, :N_SPARSE].set(router_w)
    rb = jnp.zeros((1, LANE), jnp.float32).at[0, :N_SPARSE].set(router_b)
    eb = expert_b.reshape(N_EXPERT, 1, D_MODEL)
    grid = (NTB, N_EXPERT)
    out = pl.pallas_call(
        _moe_block_kernel,
        grid=grid,
        in_specs=[
            pl.BlockSpec((BT, D_MODEL), lambda t, e: (t, 0)),
            pl.BlockSpec((1, D_MODEL, D_MODEL), lambda t, e: (e, 0, 0)),
            pl.BlockSpec((1, 1, D_MODEL), lambda t, e: (e, 0, 0)),
            pl.BlockSpec((D_MODEL, LANE), lambda t, e: (0, 0)),
            pl.BlockSpec((1, LANE), lambda t, e: (0, 0)),
        ],
        out_specs=pl.BlockSpec((BT, D_MODEL), lambda t, e: (t, 0)),
        out_shape=jax.ShapeDtypeStruct((N_TOKENS, D_MODEL), jnp.float32),
        scratch_shapes=[
            pltpu.VMEM((BT, D_MODEL), jnp.float32),
            pltpu.VMEM((BT, LANE), jnp.float32),
        ],
        compiler_params=pltpu.CompilerParams(
            dimension_semantics=("arbitrary", "arbitrary"),
        ),
    )(x, expert_w, eb, rw, rb)
    return out
